# baseline (speedup 1.0000x reference)
import jax
import jax.numpy as jnp
from jax import lax
from jax.experimental import pallas as pl
from jax.experimental.pallas import tpu as pltpu
from jax.experimental.pallas import tpu_sc as plsc

_J, _K = 8, 7
_M, _N, _D = 6, 128, 128
_MH = _M // 2


def _sc_body(x_hbm, o_hbm, buf0, buf1, l0, l1, s0, s1):
    info = plsc.get_sparse_core_info()
    nc = info.num_cores
    wid = lax.axis_index("s") * nc + lax.axis_index("c")
    bufs, lsems, ssems = (buf0, buf1), (l0, l1), (s0, s1)

    def run(tasks):
        halves = [(src, dsts, h) for (src, dsts) in tasks for h in (0, 1)]
        n = len(halves)

        def load(i):
            (b, r), _, h = halves[i]
            return pltpu.make_async_copy(
                x_hbm.at[b, r, pl.ds(h * _MH, _MH)], bufs[i % 2],
                lsems[i % 2])

        for i in range(min(2, n)):
            load(i).start()
        for i in range(n):
            buf = i % 2
            load(i).wait()
            _, dsts, h = halves[i]
            stores = [
                pltpu.make_async_copy(
                    bufs[buf], o_hbm.at[bb, j, k, pl.ds(h * _MH, _MH)],
                    ssems[buf])
                for (bb, j, k) in dsts
            ]
            for st in stores:
                st.start()
            if i + 2 < n:
                for st in stores:
                    st.wait()
                load(i + 2).start()
            else:
                for st in stores:
                    st.wait()

    def bgroup(g):
        b, k = g // 6, g % 6
        return ((b, k), [(b, j, k) for j in range(_J)])

    def diag(t):
        b, j = t // _J, t % _J
        return ((b, 6 + j), [(b, j, 6)])

    @pl.when(wid < 16)
    def _():
        run([bgroup(wid * 2), bgroup(wid * 2 + 1)])

    @pl.when(wid >= 16)
    def _():
        run([bgroup(32 + (wid - 16))]
            + [diag((wid - 16) * 4 + i) for i in range(4)])


def kernel(x):
    b, s, n, m, d = x.shape
    xt = x.transpose(0, 1, 3, 2, 4)
    mesh = plsc.VectorSubcoreMesh(core_axis_name="c", subcore_axis_name="s")
    fn = pl.kernel(
        _sc_body,
        out_type=jax.ShapeDtypeStruct((b, _J, _K, m, n, d), x.dtype),
        mesh=mesh,
        scratch_types=[
            pltpu.VMEM((_MH, n, d), x.dtype),
            pltpu.VMEM((_MH, n, d), x.dtype),
            pltpu.SemaphoreType.DMA,
            pltpu.SemaphoreType.DMA,
            pltpu.SemaphoreType.DMA,
            pltpu.SemaphoreType.DMA,
        ],
    )
    return fn(xt).transpose(0, 1, 2, 4, 3, 5)


# speedup vs baseline: 1.0733x; 1.0733x over previous
import jax
import jax.numpy as jnp
from jax import lax
from jax.experimental import pallas as pl
from jax.experimental.pallas import tpu as pltpu
from jax.experimental.pallas import tpu_sc as plsc

_J, _K = 8, 7


def _sc_body(x_hbm, o_hbm, buf, lsem, ssem):
    info = plsc.get_sparse_core_info()
    nc = info.num_cores
    wid = lax.axis_index("s") * nc + lax.axis_index("c")

    def row_copy(src, dsts):
        b, r = src
        ld = pltpu.make_async_copy(x_hbm.at[b, r], buf, lsem)
        ld.start()
        ld.wait()
        stores = [
            pltpu.make_async_copy(buf, o_hbm.at[bb, j, k], ssem)
            for (bb, j, k) in dsts
        ]
        for st in stores:
            st.start()
        for st in stores:
            st.wait()

    def bgroup(g):
        b, k = g // 6, g % 6
        row_copy((b, k), [(b, j, k) for j in range(_J)])

    def diag(t):
        b, j = t // _J, t % _J
        row_copy((b, 6 + j), [(b, j, 6)])

    @pl.when(wid < 16)
    def _():
        for i in range(2):
            bgroup(wid * 2 + i)

    @pl.when(wid >= 16)
    def _():
        bgroup(32 + (wid - 16))
        for i in range(4):
            diag((wid - 16) * 4 + i)


def kernel(x):
    b, s, n, m, d = x.shape
    xt = x.transpose(0, 1, 3, 2, 4)
    mesh = plsc.VectorSubcoreMesh(core_axis_name="c", subcore_axis_name="s")
    fn = pl.kernel(
        _sc_body,
        out_type=jax.ShapeDtypeStruct((b, _J, _K, m, n, d), x.dtype),
        mesh=mesh,
        scratch_types=[
            pltpu.VMEM((m, n, d), x.dtype),
            pltpu.SemaphoreType.DMA,
            pltpu.SemaphoreType.DMA,
        ],
    )
    return fn(xt).transpose(0, 1, 2, 4, 3, 5)
